# Initial kernel scaffold; baseline (speedup 1.0000x reference)
import jax
import jax.numpy as jnp
from jax.experimental import pallas as pl


def kernel(embedding, W_enc, W_dec, cb0, cb1, cb2):
    raise NotImplementedError("write your pallas kernel here")


# speedup vs baseline: 1.4723x; 1.4723x over previous
import jax
import jax.numpy as jnp
from jax.experimental import pallas as pl

_B, _D, _H, _K = 16384, 768, 64, 1024
_BB = 256
_NB = _B // _BB


def _rqvae_block(emb_ref, wencT_ref, wdecT_ref, cb0_ref, cb1_ref, cb2_ref,
                 cb0T_ref, cb1T_ref, cb2T_ref, scal_ref, used_ref):
    i = pl.program_id(0)
    emb = emb_ref[...]
    lat = jax.lax.dot_general(emb, wencT_ref[...], (((1,), (0,)), ((), ())),
                              preferred_element_type=jnp.float32)
    rem = lat
    acc = jnp.zeros_like(lat)
    ss_resid = jnp.float32(0.0)
    used_rows = []
    for cb_ref, cbT_ref in ((cb0_ref, cb0T_ref), (cb1_ref, cb1T_ref),
                            (cb2_ref, cb2T_ref)):
        cbT = cbT_ref[...]
        cb_norm = jnp.sum(cbT * cbT, axis=0, keepdims=True)
        prod = jax.lax.dot_general(rem, cbT, (((1,), (0,)), ((), ())),
                                   preferred_element_type=jnp.float32)
        scores = cb_norm - 2.0 * prod
        min_score = jnp.min(scores, axis=1, keepdims=True)
        kiota = jax.lax.broadcasted_iota(jnp.int32, (_BB, _K), 1)
        cand = jnp.where(scores <= min_score, kiota, _K)
        idx = jnp.min(cand, axis=1, keepdims=True)
        onehot = (kiota == idx).astype(jnp.float32)
        vecs = jax.lax.dot_general(onehot, cb_ref[...], (((1,), (0,)), ((), ())),
                                   preferred_element_type=jnp.float32)
        rem = rem - vecs
        acc = acc + vecs
        ss_resid = ss_resid + jnp.sum(rem * rem)
        used_rows.append(jnp.max(onehot, axis=0, keepdims=True))
    recon = jax.lax.dot_general(acc, wdecT_ref[...], (((1,), (0,)), ((), ())),
                                preferred_element_type=jnp.float32)
    diff = recon - emb
    ss_recon = jnp.sum(diff * diff)
    used_all = jnp.concatenate(
        used_rows + [jnp.zeros((5, _K), jnp.float32)], axis=0)

    lane = jax.lax.broadcasted_iota(jnp.int32, (1, 128), 1)
    scal_part = (jnp.where(lane == 0, ss_resid, 0.0)
                 + jnp.where(lane == 1, ss_recon, 0.0))

    @pl.when(i == 0)
    def _init():
        scal_ref[...] = jnp.zeros_like(scal_ref)
        used_ref[...] = jnp.zeros_like(used_ref)

    scal_ref[...] += scal_part
    used_ref[...] = jnp.maximum(used_ref[...], used_all)


def kernel(embedding, W_enc, W_dec, cb0, cb1, cb2):
    fullK = pl.BlockSpec((_K, _H), lambda i: (0, 0))
    fullKT = pl.BlockSpec((_H, _K), lambda i: (0, 0))
    scal, used = pl.pallas_call(
        _rqvae_block,
        grid=(_NB,),
        in_specs=[
            pl.BlockSpec((_BB, _D), lambda i: (i, 0)),
            pl.BlockSpec((_D, _H), lambda i: (0, 0)),
            pl.BlockSpec((_H, _D), lambda i: (0, 0)),
            fullK, fullK, fullK,
            fullKT, fullKT, fullKT,
        ],
        out_specs=[
            pl.BlockSpec((1, 128), lambda i: (0, 0)),
            pl.BlockSpec((8, _K), lambda i: (0, 0)),
        ],
        out_shape=[
            jax.ShapeDtypeStruct((1, 128), jnp.float32),
            jax.ShapeDtypeStruct((8, _K), jnp.float32),
        ],
    )(embedding, W_enc.T, W_dec.T, cb0, cb1, cb2, cb0.T, cb1.T, cb2.T)
    rqvae_loss = 1.25 * scal[0, 0] / (_B * _H)
    recon_loss = scal[0, 1] / (_B * _D)
    loss = recon_loss + rqvae_loss
    uniques = jnp.sum(used[:3] > 0, axis=1).astype(jnp.int32)
    return (loss, recon_loss, rqvae_loss, uniques)


# speedup vs baseline: 2.1734x; 1.4762x over previous
import jax
import jax.numpy as jnp
from jax.experimental import pallas as pl

_B, _D, _H, _K = 16384, 768, 64, 1024
_BB = 2048
_NB = _B // _BB


def _rqvae_block(emb_ref, wencT_ref, wdecT_ref, cb0_ref, cb1_ref, cb2_ref,
                 cb0T_ref, cb1T_ref, cb2T_ref, scal_ref, used_ref):
    i = pl.program_id(0)
    emb = emb_ref[...]
    lat = jax.lax.dot_general(emb, wencT_ref[...], (((1,), (0,)), ((), ())),
                              preferred_element_type=jnp.float32)
    rem = lat
    acc = jnp.zeros_like(lat)
    ss_resid = jnp.float32(0.0)
    used_rows = []
    for cb_ref, cbT_ref in ((cb0_ref, cb0T_ref), (cb1_ref, cb1T_ref),
                            (cb2_ref, cb2T_ref)):
        cbT = cbT_ref[...]
        cb_norm = jnp.sum(cbT * cbT, axis=0, keepdims=True)
        prod = jax.lax.dot_general(rem, cbT, (((1,), (0,)), ((), ())),
                                   preferred_element_type=jnp.float32)
        scores = cb_norm - 2.0 * prod
        min_score = jnp.min(scores, axis=1, keepdims=True)
        kiota = jax.lax.broadcasted_iota(jnp.int32, (_BB, _K), 1)
        cand = jnp.where(scores <= min_score, kiota, _K)
        idx = jnp.min(cand, axis=1, keepdims=True)
        onehot = (kiota == idx).astype(jnp.float32)
        vecs = jax.lax.dot_general(onehot, cb_ref[...], (((1,), (0,)), ((), ())),
                                   preferred_element_type=jnp.float32)
        rem = rem - vecs
        acc = acc + vecs
        ss_resid = ss_resid + jnp.sum(rem * rem)
        used_rows.append(jnp.max(onehot, axis=0, keepdims=True))
    recon = jax.lax.dot_general(acc, wdecT_ref[...], (((1,), (0,)), ((), ())),
                                preferred_element_type=jnp.float32)
    diff = recon - emb
    ss_recon = jnp.sum(diff * diff)
    used_all = jnp.concatenate(
        used_rows + [jnp.zeros((5, _K), jnp.float32)], axis=0)

    lane = jax.lax.broadcasted_iota(jnp.int32, (1, 128), 1)
    scal_part = (jnp.where(lane == 0, ss_resid, 0.0)
                 + jnp.where(lane == 1, ss_recon, 0.0))

    @pl.when(i == 0)
    def _init():
        scal_ref[...] = jnp.zeros_like(scal_ref)
        used_ref[...] = jnp.zeros_like(used_ref)

    scal_ref[...] += scal_part
    used_ref[...] = jnp.maximum(used_ref[...], used_all)


def kernel(embedding, W_enc, W_dec, cb0, cb1, cb2):
    fullK = pl.BlockSpec((_K, _H), lambda i: (0, 0))
    fullKT = pl.BlockSpec((_H, _K), lambda i: (0, 0))
    scal, used = pl.pallas_call(
        _rqvae_block,
        grid=(_NB,),
        in_specs=[
            pl.BlockSpec((_BB, _D), lambda i: (i, 0)),
            pl.BlockSpec((_D, _H), lambda i: (0, 0)),
            pl.BlockSpec((_H, _D), lambda i: (0, 0)),
            fullK, fullK, fullK,
            fullKT, fullKT, fullKT,
        ],
        out_specs=[
            pl.BlockSpec((1, 128), lambda i: (0, 0)),
            pl.BlockSpec((8, _K), lambda i: (0, 0)),
        ],
        out_shape=[
            jax.ShapeDtypeStruct((1, 128), jnp.float32),
            jax.ShapeDtypeStruct((8, _K), jnp.float32),
        ],
    )(embedding, W_enc.T, W_dec.T, cb0, cb1, cb2, cb0.T, cb1.T, cb2.T)
    rqvae_loss = 1.25 * scal[0, 0] / (_B * _H)
    recon_loss = scal[0, 1] / (_B * _D)
    loss = recon_loss + rqvae_loss
    uniques = jnp.sum(used[:3] > 0, axis=1).astype(jnp.int32)
    return (loss, recon_loss, rqvae_loss, uniques)


# speedup vs baseline: 2.5814x; 1.1877x over previous
import jax
import jax.numpy as jnp
from jax.experimental import pallas as pl

_B, _D, _H, _K = 16384, 768, 64, 1024
_BB = 2048
_NB = _B // _BB


def _rqvae_block(emb_ref, wencT_ref, wdecT_ref, cb0_ref, cb1_ref, cb2_ref,
                 cb0Tm2_ref, cb1Tm2_ref, cb2Tm2_ref, scal_ref, used_ref):
    i = pl.program_id(0)
    emb = emb_ref[...]
    lat = jax.lax.dot_general(emb, wencT_ref[...], (((1,), (0,)), ((), ())),
                              preferred_element_type=jnp.float32)
    rem = lat
    acc = jnp.zeros_like(lat)
    ss_resid = jnp.float32(0.0)
    used_rows = []
    kiota = jax.lax.broadcasted_iota(jnp.int32, (_BB, _K), 1)
    for cb_ref, cbTm2_ref in ((cb0_ref, cb0Tm2_ref), (cb1_ref, cb1Tm2_ref),
                              (cb2_ref, cb2Tm2_ref)):
        cbTm2 = cbTm2_ref[...]
        cb_norm = 0.25 * jnp.sum(cbTm2 * cbTm2, axis=0, keepdims=True)
        prod = jax.lax.dot_general(rem, cbTm2, (((1,), (0,)), ((), ())),
                                   preferred_element_type=jnp.float32)
        scores = cb_norm + prod
        bits = jax.lax.bitcast_convert_type(scores, jnp.int32)
        packed = jax.lax.bitcast_convert_type(
            jnp.bitwise_or(jnp.bitwise_and(bits, jnp.int32(-1024)), kiota),
            jnp.float32)
        minv = jnp.min(packed, axis=1, keepdims=True)
        onehot = (packed == minv).astype(jnp.float32)
        vecs = jax.lax.dot_general(onehot, cb_ref[...], (((1,), (0,)), ((), ())),
                                   preferred_element_type=jnp.float32)
        rem = rem - vecs
        acc = acc + vecs
        ss_resid = ss_resid + jnp.sum(rem * rem)
        used_rows.append(jnp.max(onehot, axis=0, keepdims=True))
    recon = jax.lax.dot_general(acc, wdecT_ref[...], (((1,), (0,)), ((), ())),
                                preferred_element_type=jnp.float32)
    diff = recon - emb
    ss_recon = jnp.sum(diff * diff)
    used_all = jnp.concatenate(
        used_rows + [jnp.zeros((5, _K), jnp.float32)], axis=0)

    lane = jax.lax.broadcasted_iota(jnp.int32, (1, 128), 1)
    scal_part = (jnp.where(lane == 0, ss_resid, 0.0)
                 + jnp.where(lane == 1, ss_recon, 0.0))

    @pl.when(i == 0)
    def _init():
        scal_ref[...] = jnp.zeros_like(scal_ref)
        used_ref[...] = jnp.zeros_like(used_ref)

    scal_ref[...] += scal_part
    used_ref[...] = jnp.maximum(used_ref[...], used_all)


def kernel(embedding, W_enc, W_dec, cb0, cb1, cb2):
    fullK = pl.BlockSpec((_K, _H), lambda i: (0, 0))
    fullKT = pl.BlockSpec((_H, _K), lambda i: (0, 0))
    scal, used = pl.pallas_call(
        _rqvae_block,
        grid=(_NB,),
        in_specs=[
            pl.BlockSpec((_BB, _D), lambda i: (i, 0)),
            pl.BlockSpec((_D, _H), lambda i: (0, 0)),
            pl.BlockSpec((_H, _D), lambda i: (0, 0)),
            fullK, fullK, fullK,
            fullKT, fullKT, fullKT,
        ],
        out_specs=[
            pl.BlockSpec((1, 128), lambda i: (0, 0)),
            pl.BlockSpec((8, _K), lambda i: (0, 0)),
        ],
        out_shape=[
            jax.ShapeDtypeStruct((1, 128), jnp.float32),
            jax.ShapeDtypeStruct((8, _K), jnp.float32),
        ],
    )(embedding, W_enc.T, W_dec.T, cb0, cb1, cb2,
      -2.0 * cb0.T, -2.0 * cb1.T, -2.0 * cb2.T)
    rqvae_loss = 1.25 * scal[0, 0] / (_B * _H)
    recon_loss = scal[0, 1] / (_B * _D)
    loss = recon_loss + rqvae_loss
    uniques = jnp.sum(used[:3] > 0, axis=1).astype(jnp.int32)
    return (loss, recon_loss, rqvae_loss, uniques)


# speedup vs baseline: 2.7889x; 1.0804x over previous
import jax
import jax.numpy as jnp
from jax.experimental import pallas as pl

_B, _D, _H, _K = 16384, 768, 64, 1024
_BB = 2048
_NB = _B // _BB


def _rqvae_block(emb_ref, wencT_ref, wdec_ref, gram_ref, cb0_ref, cb1_ref, cb2_ref,
                 cb0Tm2_ref, cb1Tm2_ref, cb2Tm2_ref, scal_ref, used_ref):
    i = pl.program_id(0)
    emb = emb_ref[...]
    lat = jax.lax.dot_general(emb, wencT_ref[...], (((1,), (0,)), ((), ())),
                              preferred_element_type=jnp.float32)
    rem = lat
    acc = jnp.zeros_like(lat)
    ss_resid = jnp.float32(0.0)
    used_rows = []
    kiota = jax.lax.broadcasted_iota(jnp.int32, (_BB, _K), 1)
    for cb_ref, cbTm2_ref in ((cb0_ref, cb0Tm2_ref), (cb1_ref, cb1Tm2_ref),
                              (cb2_ref, cb2Tm2_ref)):
        cbTm2 = cbTm2_ref[...]
        cb_norm = 0.25 * jnp.sum(cbTm2 * cbTm2, axis=0, keepdims=True)
        prod = jax.lax.dot_general(rem, cbTm2, (((1,), (0,)), ((), ())),
                                   preferred_element_type=jnp.float32)
        scores = cb_norm + prod
        bits = jax.lax.bitcast_convert_type(scores, jnp.int32)
        packed = jax.lax.bitcast_convert_type(
            jnp.bitwise_or(jnp.bitwise_and(bits, jnp.int32(-1024)), kiota),
            jnp.float32)
        minv = jnp.min(packed, axis=1, keepdims=True)
        onehot = (packed == minv).astype(jnp.float32)
        vecs = jax.lax.dot_general(onehot, cb_ref[...], (((1,), (0,)), ((), ())),
                                   preferred_element_type=jnp.float32)
        rem = rem - vecs
        acc = acc + vecs
        ss_resid = ss_resid + jnp.sum(rem * rem)
        used_rows.append(jnp.max(onehot, axis=0, keepdims=True))
    t = jax.lax.dot_general(emb, wdec_ref[...], (((1,), (0,)), ((), ())),
                            preferred_element_type=jnp.float32)
    q = jax.lax.dot_general(acc, gram_ref[...], (((1,), (0,)), ((), ())),
                            preferred_element_type=jnp.float32)
    ss_recon = jnp.sum((q - 2.0 * t) * acc) + jnp.sum(emb * emb)
    used_all = jnp.concatenate(
        used_rows + [jnp.zeros((5, _K), jnp.float32)], axis=0)

    lane = jax.lax.broadcasted_iota(jnp.int32, (1, 128), 1)
    scal_part = (jnp.where(lane == 0, ss_resid, 0.0)
                 + jnp.where(lane == 1, ss_recon, 0.0))

    @pl.when(i == 0)
    def _init():
        scal_ref[...] = jnp.zeros_like(scal_ref)
        used_ref[...] = jnp.zeros_like(used_ref)

    scal_ref[...] += scal_part
    used_ref[...] = jnp.maximum(used_ref[...], used_all)


def kernel(embedding, W_enc, W_dec, cb0, cb1, cb2):
    fullK = pl.BlockSpec((_K, _H), lambda i: (0, 0))
    fullKT = pl.BlockSpec((_H, _K), lambda i: (0, 0))
    scal, used = pl.pallas_call(
        _rqvae_block,
        grid=(_NB,),
        in_specs=[
            pl.BlockSpec((_BB, _D), lambda i: (i, 0)),
            pl.BlockSpec((_D, _H), lambda i: (0, 0)),
            pl.BlockSpec((_D, _H), lambda i: (0, 0)),
            pl.BlockSpec((_H, _H), lambda i: (0, 0)),
            fullK, fullK, fullK,
            fullKT, fullKT, fullKT,
        ],
        out_specs=[
            pl.BlockSpec((1, 128), lambda i: (0, 0)),
            pl.BlockSpec((8, _K), lambda i: (0, 0)),
        ],
        out_shape=[
            jax.ShapeDtypeStruct((1, 128), jnp.float32),
            jax.ShapeDtypeStruct((8, _K), jnp.float32),
        ],
    )(embedding, W_enc.T, W_dec, W_dec.T @ W_dec, cb0, cb1, cb2,
      -2.0 * cb0.T, -2.0 * cb1.T, -2.0 * cb2.T)
    rqvae_loss = 1.25 * scal[0, 0] / (_B * _H)
    recon_loss = scal[0, 1] / (_B * _D)
    loss = recon_loss + rqvae_loss
    uniques = jnp.sum(used[:3] > 0, axis=1).astype(jnp.int32)
    return (loss, recon_loss, rqvae_loss, uniques)
